# baseline (speedup 1.0000x reference)
import functools

import jax
import jax.numpy as jnp
from jax.experimental import pallas as pl
from jax.experimental.pallas import tpu as pltpu

B, T = 16384, 100
D = 128
N = B * T
NUM_ROWS = 5
BLK = 1024
NBLK = N // BLK


def _tc_body(tok_ref, table_ref, out_ref):
    tok = jnp.reshape(tok_ref[...], (BLK, 1))
    acc = jnp.broadcast_to(table_ref[0][None, :], (BLK, D))
    for k in range(1, NUM_ROWS):
        acc = jnp.where(tok == k, table_ref[k][None, :], acc)
    out_ref[...] = acc


@functools.partial(jax.jit)
def _tc_gather(idx3, table):
    return pl.pallas_call(
        _tc_body,
        grid=(NBLK,),
        in_specs=[
            pl.BlockSpec((1, 1, BLK), lambda i: (i, 0, 0)),
            pl.BlockSpec((NUM_ROWS, D), lambda i: (0, 0)),
        ],
        out_specs=pl.BlockSpec((BLK, D), lambda i: (i, 0)),
        out_shape=jax.ShapeDtypeStruct((N, D), jnp.float32),
    )(idx3, table)


def kernel(token_types, table):
    idx3 = jnp.reshape(token_types, (NBLK, 1, BLK)).astype(jnp.int32)
    out = _tc_gather(idx3, table)
    return jnp.reshape(out, (B, T, D))


# speedup vs baseline: 1.3862x; 1.3862x over previous
import functools

import jax
import jax.numpy as jnp
from jax.experimental import pallas as pl
from jax.experimental.pallas import tpu as pltpu

B, T = 16384, 100
D = 128
N = B * T
NUM_ROWS = 5
BLK = 8192
W = BLK // 8
NBLK = N // BLK


def _tc_body(tok_ref, table_ref, out_ref):
    for s in range(8):
        tok = jnp.reshape(tok_ref[0, s, :], (W, 1))
        acc = jnp.broadcast_to(table_ref[0][None, :], (W, D))
        for k in range(1, NUM_ROWS):
            acc = jnp.where(tok == k, table_ref[k][None, :], acc)
        out_ref[pl.ds(s * W, W), :] = acc


@functools.partial(jax.jit)
def _tc_gather(idx3, table):
    return pl.pallas_call(
        _tc_body,
        grid=(NBLK,),
        in_specs=[
            pl.BlockSpec((1, 8, W), lambda i: (i, 0, 0)),
            pl.BlockSpec((NUM_ROWS, D), lambda i: (0, 0)),
        ],
        out_specs=pl.BlockSpec((BLK, D), lambda i: (i, 0)),
        out_shape=jax.ShapeDtypeStruct((N, D), jnp.float32),
    )(idx3, table)


def kernel(token_types, table):
    idx3 = jnp.reshape(token_types, (NBLK, 8, W)).astype(jnp.int32)
    out = _tc_gather(idx3, table)
    return jnp.reshape(out, (B, T, D))
